# Initial kernel scaffold; baseline (speedup 1.0000x reference)
import jax
import jax.numpy as jnp
import numpy as np
from jax.experimental import pallas as pl
from jax.experimental.pallas import tpu as pltpu
from jax.experimental.pallas import tpu_sc as plsc

_S = 1024
_N = 50
_NMAX = 64
_D = 64
_EPS = 0.1
_E_FAC = np.float32(1.0 / np.sqrt(1.0 + _EPS ** 2))
_C_NOISE = np.float32(_EPS / np.sqrt(_D))
_P = 2 * _NMAX + 1
_T = 2 * _N + 1

_WIN = 128


def _sc_gather_pair(mus_class, mus_label, idx_cls, idx_lab):
    n_cls = idx_cls.size
    n_lab = idx_lab.size
    mesh = plsc.VectorSubcoreMesh(core_axis_name="c", subcore_axis_name="s")

    @pl.kernel(
        out_type=(
            jax.ShapeDtypeStruct((n_cls, _D), jnp.float32),
            jax.ShapeDtypeStruct((n_lab, _D), jnp.float32),
        ),
        mesh=mesh,
    )
    def k(cls_hbm, lab_hbm, ic_hbm, il_hbm, oc_hbm, ol_hbm):
        def body_cls(i_vmem, o_vmem):
            pltpu.sync_copy(cls_hbm.at[i_vmem.at[0]], o_vmem)

        pltpu.emit_pipeline(
            body_cls,
            grid=(n_cls // _WIN,),
            in_specs=[pl.BlockSpec((1, _WIN), lambda i: (0, i))],
            out_specs=[pl.BlockSpec((_WIN, _D), lambda i: (i, 0))],
            core_axis_name=("c", "s"),
            dimension_semantics=(pltpu.PARALLEL,),
        )(ic_hbm, oc_hbm)

        def body_lab(i_vmem, o_vmem):
            pltpu.sync_copy(lab_hbm.at[i_vmem.at[0]], o_vmem)

        pltpu.emit_pipeline(
            body_lab,
            grid=(n_lab // _WIN,),
            in_specs=[pl.BlockSpec((1, _WIN), lambda i: (0, i))],
            out_specs=[pl.BlockSpec((_WIN, _D), lambda i: (i, 0))],
            core_axis_name=("c", "s"),
            dimension_semantics=(pltpu.PARALLEL,),
        )(il_hbm, ol_hbm)

    return k(mus_class, mus_label,
             idx_cls.reshape(1, n_cls), idx_lab.reshape(1, n_lab))


_B = 8


def _assemble_body(shift_ref, cls_ref, lab_ref, nc_ref, nq_ref, out_ref):
    cls = cls_ref[...]
    lab = lab_ref[...]
    nc = nc_ref[...]
    nq = nq_ref[...]
    ctx = _E_FAC * (cls[:, :_N, :] + _C_NOISE * nc)
    q = _E_FAC * (cls[:, _N:, :] + _C_NOISE * nq)
    pair = jnp.stack([ctx, lab], axis=2).reshape(_B, 2 * _N, _D)
    feat = jnp.concatenate([pair, q], axis=1)
    sh = shift_ref[...]
    row = jax.lax.broadcasted_iota(jnp.int32, (_B, _T, _P), 1)
    col = jax.lax.broadcasted_iota(jnp.int32, (_B, _T, _P), 2)
    pos = (col == sh + row).astype(jnp.float32)
    out_ref[...] = jnp.concatenate([pos, feat], axis=2)


def _assemble(shifts, cls3, lab3, noise_ctx, noise_q):
    grid = (_S // _B,)
    return pl.pallas_call(
        _assemble_body,
        grid=grid,
        in_specs=[
            pl.BlockSpec((_B, 1, 1), lambda i: (i, 0, 0)),
            pl.BlockSpec((_B, _N + 1, _D), lambda i: (i, 0, 0)),
            pl.BlockSpec((_B, _N, _D), lambda i: (i, 0, 0)),
            pl.BlockSpec((_B, _N, _D), lambda i: (i, 0, 0)),
            pl.BlockSpec((_B, 1, _D), lambda i: (i, 0, 0)),
        ],
        out_specs=pl.BlockSpec((_B, _T, _P + _D), lambda i: (i, 0, 0)),
        out_shape=jax.ShapeDtypeStruct((_S, _T, _P + _D), jnp.float32),
    )(shifts.reshape(_S, 1, 1), cls3, lab3, noise_ctx, noise_q)


def kernel(example, label, noise_ctx, noise_q, shifts, mus_label, mus_class):
    idx_cls = example.reshape(-1).astype(jnp.int32)
    idx_lab = label[:, :_N].reshape(-1).astype(jnp.int32)
    cls_flat, lab_flat = _sc_gather_pair(mus_class, mus_label, idx_cls, idx_lab)
    cls3 = cls_flat.reshape(_S, _N + 1, _D)
    lab3 = lab_flat.reshape(_S, _N, _D)
    return _assemble(shifts.astype(jnp.int32), cls3, lab3,
                     noise_ctx, noise_q.reshape(_S, 1, _D))


# speedup vs baseline: 31.3958x; 31.3958x over previous
import jax
import jax.numpy as jnp
import numpy as np
from jax.experimental import pallas as pl
from jax.experimental.pallas import tpu as pltpu
from jax.experimental.pallas import tpu_sc as plsc

_S = 1024
_N = 50
_NMAX = 64
_D = 64
_EPS = 0.1
_E_FAC = np.float32(1.0 / np.sqrt(1.0 + _EPS ** 2))
_C_NOISE = np.float32(_EPS / np.sqrt(_D))
_P = 2 * _NMAX + 1
_T = 2 * _N + 1

_WIN = 128


_NW = 32


def _sc_gather_pair(mus_class, mus_label, idx_cls, idx_lab):
    n_cls = idx_cls.size
    n_lab = idx_lab.size
    w_cls = n_cls // _WIN
    w_lab = n_lab // _WIN
    j_cls = -(-w_cls // _NW)
    j_lab = -(-w_lab // _NW)
    mesh = plsc.VectorSubcoreMesh(core_axis_name="c", subcore_axis_name="s")
    cls_wide = mus_class.reshape(-1, 2 * _D)
    lab_wide = mus_label.reshape(-1, 2 * _D)

    @pl.kernel(
        out_type=(
            jax.ShapeDtypeStruct((n_cls, 2 * _D), jnp.float32),
            jax.ShapeDtypeStruct((n_lab, 2 * _D), jnp.float32),
        ),
        mesh=mesh,
        scratch_types=[
            pltpu.VMEM((_WIN,), jnp.int32),
            pltpu.VMEM((_WIN,), jnp.int32),
            pltpu.VMEM((_WIN, 2 * _D), jnp.float32),
            pltpu.SemaphoreType.DMA,
        ],
    )
    def k(cls_hbm, lab_hbm, ic_hbm, il_hbm, oc_hbm, ol_hbm,
          idx_v, half_v, rows_v, sem):
        wid = jax.lax.axis_index("s") * 2 + jax.lax.axis_index("c")

        def one_window(tab_hbm, i_hbm, o_hbm, w):
            base = w * _WIN
            pltpu.sync_copy(i_hbm.at[pl.ds(base, _WIN)], idx_v)
            for c in range(_WIN // 16):
                sl = pl.ds(c * 16, 16)
                half_v[sl] = jax.lax.shift_right_logical(idx_v[sl], 1)
            pltpu.async_copy(tab_hbm.at[half_v], rows_v, sem).wait()
            pltpu.sync_copy(rows_v, o_hbm.at[pl.ds(base, _WIN)])

        for j in range(j_cls):
            w = wid + j * _NW

            @pl.when(w < w_cls)
            def _():
                one_window(cls_hbm, ic_hbm, oc_hbm, w)

        for j in range(j_lab):
            w = wid + j * _NW

            @pl.when(w < w_lab)
            def _():
                one_window(lab_hbm, il_hbm, ol_hbm, w)

    return k(cls_wide, lab_wide, idx_cls, idx_lab)


_B = 8


def _assemble_body(shift_ref, ex_ref, lb_ref, cls_ref, lab_ref,
                   nc_ref, nq_ref, out_ref):
    clsw = cls_ref[...]
    labw = lab_ref[...]
    exp = ex_ref[...] & 1
    lbp = lb_ref[...] & 1
    cls = jnp.where(exp == 1, clsw[:, :, _D:], clsw[:, :, :_D])
    lab = jnp.where(lbp == 1, labw[:, :, _D:], labw[:, :, :_D])
    nc = nc_ref[...]
    nq = nq_ref[...]
    ctx = _E_FAC * (cls[:, :_N, :] + _C_NOISE * nc)
    q = _E_FAC * (cls[:, _N:, :] + _C_NOISE * nq)
    pair = jnp.stack([ctx, lab], axis=2).reshape(_B, 2 * _N, _D)
    feat = jnp.concatenate([pair, q], axis=1)
    sh = shift_ref[...]
    row = jax.lax.broadcasted_iota(jnp.int32, (_B, _T, _P), 1)
    col = jax.lax.broadcasted_iota(jnp.int32, (_B, _T, _P), 2)
    pos = (col == sh + row).astype(jnp.float32)
    out_ref[...] = jnp.concatenate([pos, feat], axis=2)


def _assemble(shifts, example3, label3, cls3, lab3, noise_ctx, noise_q):
    grid = (_S // _B,)
    return pl.pallas_call(
        _assemble_body,
        grid=grid,
        in_specs=[
            pl.BlockSpec((_B, 1, 1), lambda i: (i, 0, 0)),
            pl.BlockSpec((_B, _N + 1, 1), lambda i: (i, 0, 0)),
            pl.BlockSpec((_B, _N, 1), lambda i: (i, 0, 0)),
            pl.BlockSpec((_B, _N + 1, 2 * _D), lambda i: (i, 0, 0)),
            pl.BlockSpec((_B, _N, 2 * _D), lambda i: (i, 0, 0)),
            pl.BlockSpec((_B, _N, _D), lambda i: (i, 0, 0)),
            pl.BlockSpec((_B, 1, _D), lambda i: (i, 0, 0)),
        ],
        out_specs=pl.BlockSpec((_B, _T, _P + _D), lambda i: (i, 0, 0)),
        out_shape=jax.ShapeDtypeStruct((_S, _T, _P + _D), jnp.float32),
    )(shifts.reshape(_S, 1, 1), example3, label3, cls3, lab3,
      noise_ctx, noise_q)


def kernel(example, label, noise_ctx, noise_q, shifts, mus_label, mus_class):
    example = example.astype(jnp.int32)
    label = label.astype(jnp.int32)
    idx_cls = example.reshape(-1)
    idx_lab = label[:, :_N].reshape(-1)
    cls_flat, lab_flat = _sc_gather_pair(mus_class, mus_label, idx_cls, idx_lab)
    cls3 = cls_flat.reshape(_S, _N + 1, 2 * _D)
    lab3 = lab_flat.reshape(_S, _N, 2 * _D)
    return _assemble(shifts.astype(jnp.int32),
                     example.reshape(_S, _N + 1, 1),
                     label[:, :_N].reshape(_S, _N, 1),
                     cls3, lab3, noise_ctx, noise_q.reshape(_S, 1, _D))


# speedup vs baseline: 34.3016x; 1.0926x over previous
import jax
import jax.numpy as jnp
import numpy as np
from jax.experimental import pallas as pl
from jax.experimental.pallas import tpu as pltpu
from jax.experimental.pallas import tpu_sc as plsc

_S = 1024
_N = 50
_NMAX = 64
_D = 64
_EPS = 0.1
_E_FAC = np.float32(1.0 / np.sqrt(1.0 + _EPS ** 2))
_C_NOISE = np.float32(_EPS / np.sqrt(_D))
_P = 2 * _NMAX + 1
_T = 2 * _N + 1

_NW = 32
_C = 4
_SC = _S // _C
_RC = _SC * (_N + 1) // _NW
_RL = _SC * _N // _NW
_B = 16


def _pieces(r):
    out = []
    off = 0
    while off < r:
        sz = min(128, r - off)
        out.append((off, sz))
        off += sz
    return out


def _ceil16(r):
    return -(-r // 16)


def _sc_gather_chunk(k, cls_wide, lab_wide, idx_cls, idx_lab):
    base_cls = k * _SC * (_N + 1)
    base_lab = k * _SC * _N
    mesh = plsc.VectorSubcoreMesh(core_axis_name="c", subcore_axis_name="s")
    ib = _ceil16(_RC) * 16

    @pl.kernel(
        out_type=(
            jax.ShapeDtypeStruct((_SC * (_N + 1), 2 * _D), jnp.float32),
            jax.ShapeDtypeStruct((_SC * _N, 2 * _D), jnp.float32),
        ),
        mesh=mesh,
        scratch_types=[
            pltpu.VMEM((ib,), jnp.int32),
            pltpu.VMEM((ib,), jnp.int32),
            pltpu.VMEM((ib,), jnp.int32),
            pltpu.VMEM((ib,), jnp.int32),
            pltpu.VMEM((_RC, 2 * _D), jnp.float32),
            pltpu.VMEM((_RL, 2 * _D), jnp.float32),
            pltpu.SemaphoreType.DMA,
            pltpu.SemaphoreType.DMA,
            pltpu.SemaphoreType.DMA,
        ],
    )
    def k_fn(cls_hbm, lab_hbm, ic_hbm, il_hbm, oc_hbm, ol_hbm,
             idxc_v, halfc_v, idxl_v, halfl_v, stc_v, stl_v,
             gsem, ocsem, olsem):
        wid = jax.lax.axis_index("s") * 2 + jax.lax.axis_index("c")

        def load_shift(i_hbm, base, r, idx_v, half_v):
            pltpu.sync_copy(i_hbm.at[pl.ds(base + wid * r, r)],
                            idx_v.at[pl.ds(0, r)])
            for c in range(_ceil16(r)):
                sl = pl.ds(c * 16, 16)
                half_v[sl] = jax.lax.shift_right_logical(idx_v[sl], 1)

        def fire_gathers(tab, half_v, st_v, r):
            return [
                pltpu.async_copy(tab.at[half_v.at[pl.ds(off, sz)]],
                                 st_v.at[pl.ds(off, sz)], gsem)
                for off, sz in _pieces(r)
            ]

        load_shift(ic_hbm, base_cls, _RC, idxc_v, halfc_v)
        hc = fire_gathers(cls_hbm, halfc_v, stc_v, _RC)
        load_shift(il_hbm, base_lab, _RL, idxl_v, halfl_v)
        for h in hc:
            h.wait()
        oc = pltpu.async_copy(stc_v, oc_hbm.at[pl.ds(wid * _RC, _RC)], ocsem)
        hl = fire_gathers(lab_hbm, halfl_v, stl_v, _RL)
        for h in hl:
            h.wait()
        ol = pltpu.async_copy(stl_v, ol_hbm.at[pl.ds(wid * _RL, _RL)], olsem)
        oc.wait()
        ol.wait()

    return k_fn(cls_wide, lab_wide, idx_cls, idx_lab)


def _assemble_body(shift_ref, ex_ref, lb_ref, cls_ref, lab_ref,
                   nc_ref, nq_ref, out_ref):
    clsw = cls_ref[...]
    labw = lab_ref[...]
    exp = ex_ref[...] & 1
    lbp = lb_ref[...] & 1
    cls = jnp.where(exp == 1, clsw[:, :, _D:], clsw[:, :, :_D])
    lab = jnp.where(lbp == 1, labw[:, :, _D:], labw[:, :, :_D])
    nc = nc_ref[...]
    nq = nq_ref[...]
    ctx = _E_FAC * (cls[:, :_N, :] + _C_NOISE * nc)
    q = _E_FAC * (cls[:, _N:, :] + _C_NOISE * nq)
    pair = jnp.stack([ctx, lab], axis=2).reshape(_B, 2 * _N, _D)
    feat = jnp.concatenate([pair, q], axis=1)
    sh = shift_ref[...]
    row = jax.lax.broadcasted_iota(jnp.int32, (_B, _T, _P + _D), 1)
    col = jax.lax.broadcasted_iota(jnp.int32, (_B, _T, _P + _D), 2)
    pos = (col == sh + row).astype(jnp.float32)
    out_ref[...] = jnp.concatenate([pos[:, :, :_P], feat], axis=2)


def _assemble_body_aliased(shift_ref, ex_ref, lb_ref, cls_ref, lab_ref,
                           nc_ref, nq_ref, prev_ref, out_ref):
    del prev_ref
    _assemble_body(shift_ref, ex_ref, lb_ref, cls_ref, lab_ref,
                   nc_ref, nq_ref, out_ref)


def _assemble_chunk(k, prev, shifts3, example3, label3, cls3, lab3,
                    noise_ctx, noise_q3):
    base = k * (_SC // _B)
    grid = (_SC // _B,)

    def full(i):
        return (base + i, 0, 0)

    def local(i):
        return (i, 0, 0)

    in_specs = [
        pl.BlockSpec((_B, 1, 1), full),
        pl.BlockSpec((_B, _N + 1, 1), full),
        pl.BlockSpec((_B, _N, 1), full),
        pl.BlockSpec((_B, _N + 1, 2 * _D), local),
        pl.BlockSpec((_B, _N, 2 * _D), local),
        pl.BlockSpec((_B, _N, _D), full),
        pl.BlockSpec((_B, 1, _D), full),
    ]
    args = [shifts3, example3, label3, cls3, lab3, noise_ctx, noise_q3]
    body = _assemble_body
    io_aliases = {}
    if prev is not None:
        in_specs.append(pl.BlockSpec(memory_space=pl.ANY))
        args.append(prev)
        body = _assemble_body_aliased
        io_aliases = {7: 0}
    return pl.pallas_call(
        body,
        grid=grid,
        in_specs=in_specs,
        out_specs=pl.BlockSpec((_B, _T, _P + _D), full),
        out_shape=jax.ShapeDtypeStruct((_S, _T, _P + _D), jnp.float32),
        input_output_aliases=io_aliases,
    )(*args)


def kernel(example, label, noise_ctx, noise_q, shifts, mus_label, mus_class):
    example = example.astype(jnp.int32)
    label = label.astype(jnp.int32)
    idx_cls = example.reshape(-1)
    idx_lab = label[:, :_N].reshape(-1)
    cls_wide = mus_class.reshape(-1, 2 * _D)
    lab_wide = mus_label.reshape(-1, 2 * _D)
    shifts3 = shifts.astype(jnp.int32).reshape(_S, 1, 1)
    example3 = example.reshape(_S, _N + 1, 1)
    label3 = label[:, :_N].reshape(_S, _N, 1)
    noise_q3 = noise_q.reshape(_S, 1, _D)

    out = None
    for k in range(_C):
        cls_flat, lab_flat = _sc_gather_chunk(k, cls_wide, lab_wide,
                                              idx_cls, idx_lab)
        cls3 = cls_flat.reshape(_SC, _N + 1, 2 * _D)
        lab3 = lab_flat.reshape(_SC, _N, 2 * _D)
        out = _assemble_chunk(k, out, shifts3, example3, label3, cls3, lab3,
                              noise_ctx, noise_q3)
    return out


# speedup vs baseline: 35.3044x; 1.0292x over previous
import jax
import jax.numpy as jnp
import numpy as np
from jax.experimental import pallas as pl
from jax.experimental.pallas import tpu as pltpu
from jax.experimental.pallas import tpu_sc as plsc

_S = 1024
_N = 50
_NMAX = 64
_D = 64
_EPS = 0.1
_E_FAC = np.float32(1.0 / np.sqrt(1.0 + _EPS ** 2))
_C_NOISE = np.float32(_EPS / np.sqrt(_D))
_P = 2 * _NMAX + 1
_T = 2 * _N + 1

_NW = 32
_C = 4
_SCH = _S // _C
_SPW = _SCH // _NW
_RC = _SPW * (_N + 1)
_RL = _SPW * _N
_RPAD = 56
_B = 16


def _pieces(r):
    out = []
    off = 0
    while off < r:
        sz = min(128, r - off)
        out.append((off, sz))
        off += sz
    return out


def _sc_gather_chunk(k, cls_ov, lab_ov, idx_cls, idx_lab):
    base_cls = k * _SCH * (_N + 1)
    base_lab = k * _SCH * _N
    mesh = plsc.VectorSubcoreMesh(core_axis_name="c", subcore_axis_name="s")

    @pl.kernel(
        out_type=(
            jax.ShapeDtypeStruct((_SCH, _RPAD, 2 * _D), jnp.float32),
            jax.ShapeDtypeStruct((_SCH, _RPAD, 2 * _D), jnp.float32),
        ),
        mesh=mesh,
        scratch_types=[
            pltpu.VMEM((_RC,), jnp.int32),
            pltpu.VMEM((_RL,), jnp.int32),
            pltpu.VMEM((_RC + 8, 2 * _D), jnp.float32),
            pltpu.VMEM((_RL + 8, 2 * _D), jnp.float32),
            pltpu.SemaphoreType.DMA,
            pltpu.SemaphoreType.DMA,
        ],
    )
    def k_fn(cls_hbm, lab_hbm, ic_hbm, il_hbm, oc_hbm, ol_hbm,
             idxc_v, idxl_v, stc_v, stl_v, gsem, osem):
        wid = jax.lax.axis_index("s") * 2 + jax.lax.axis_index("c")
        s0 = wid * _SPW

        def fire_gathers(tab, idx_v, st_v, r):
            return [
                pltpu.async_copy(tab.at[idx_v.at[pl.ds(off, sz)]],
                                 st_v.at[pl.ds(off, sz)], gsem)
                for off, sz in _pieces(r)
            ]

        def copy_out(st_v, o_hbm, rows):
            return [
                pltpu.async_copy(st_v.at[pl.ds(t * rows, _RPAD)],
                                 o_hbm.at[s0 + t], osem)
                for t in range(_SPW)
            ]

        pltpu.sync_copy(ic_hbm.at[pl.ds(base_cls + wid * _RC, _RC)], idxc_v)
        hc = fire_gathers(cls_hbm, idxc_v, stc_v, _RC)
        pltpu.sync_copy(il_hbm.at[pl.ds(base_lab + wid * _RL, _RL)], idxl_v)
        hl = fire_gathers(lab_hbm, idxl_v, stl_v, _RL)
        for h in hc:
            h.wait()
        oc = copy_out(stc_v, oc_hbm, _N + 1)
        for h in hl:
            h.wait()
        ol = copy_out(stl_v, ol_hbm, _N)
        for h in oc + ol:
            h.wait()

    return k_fn(cls_ov, lab_ov, idx_cls, idx_lab)


def _assemble_body(shift_ref, cls_ref, lab_ref, nc_ref, nq_ref, out_ref):
    cls = cls_ref[...][:, :_N + 1, :_D]
    lab = lab_ref[...][:, :_N, :_D]
    nc = nc_ref[...]
    nq = nq_ref[...].reshape(_B, 1, _D)
    ctx = _E_FAC * (cls[:, :_N, :] + _C_NOISE * nc)
    q = _E_FAC * (cls[:, _N:, :] + _C_NOISE * nq)
    pair = jnp.stack([ctx, lab], axis=2).reshape(_B, 2 * _N, _D)
    feat = jnp.concatenate([pair, q], axis=1)
    sh = shift_ref[...].reshape(_B, 1, 1)
    row = jax.lax.broadcasted_iota(jnp.int32, (_B, _T, _P), 1)
    col = jax.lax.broadcasted_iota(jnp.int32, (_B, _T, _P), 2)
    pos = (col == sh + row).astype(jnp.float32)
    out_ref[...] = jnp.concatenate([pos, feat], axis=2)


def _assemble_body_aliased(shift_ref, cls_ref, lab_ref, nc_ref, nq_ref,
                           prev_ref, out_ref):
    del prev_ref
    _assemble_body(shift_ref, cls_ref, lab_ref, nc_ref, nq_ref, out_ref)


def _assemble_chunk(k, prev, shifts2, cls3, lab3, noise_ctx, noise_q):
    base = k * (_SCH // _B)
    grid = (_SCH // _B,)

    def full(i):
        return (base + i, 0, 0)

    def full2(i):
        return (base + i, 0)

    def local(i):
        return (i, 0, 0)

    in_specs = [
        pl.BlockSpec((_B, 1), full2),
        pl.BlockSpec((_B, _RPAD, 2 * _D), local),
        pl.BlockSpec((_B, _RPAD, 2 * _D), local),
        pl.BlockSpec((_B, _N, _D), full),
        pl.BlockSpec((_B, _D), full2),
    ]
    args = [shifts2, cls3, lab3, noise_ctx, noise_q]
    body = _assemble_body
    io_aliases = {}
    if prev is not None:
        in_specs.append(pl.BlockSpec(memory_space=pl.ANY))
        args.append(prev)
        body = _assemble_body_aliased
        io_aliases = {5: 0}
    return pl.pallas_call(
        body,
        grid=grid,
        in_specs=in_specs,
        out_specs=pl.BlockSpec((_B, _T, _P + _D), full),
        out_shape=jax.ShapeDtypeStruct((_S, _T, _P + _D), jnp.float32),
        input_output_aliases=io_aliases,
    )(*args)


def kernel(example, label, noise_ctx, noise_q, shifts, mus_label, mus_class):
    idx_cls = example.astype(jnp.int32).reshape(-1)
    idx_lab = label.astype(jnp.int32)[:, :_N].reshape(-1)
    cls_ov = jnp.concatenate([mus_class, jnp.roll(mus_class, -1, axis=0)], 1)
    lab_ov = jnp.concatenate([mus_label, jnp.roll(mus_label, -1, axis=0)], 1)
    shifts2 = shifts.astype(jnp.int32).reshape(_S, 1)

    out = None
    for k in range(_C):
        cls3, lab3 = _sc_gather_chunk(k, cls_ov, lab_ov, idx_cls, idx_lab)
        out = _assemble_chunk(k, out, shifts2, cls3, lab3,
                              noise_ctx, noise_q)
    return out
